# baseline (speedup 1.0000x reference)
import functools

import jax
import jax.numpy as jnp
from jax import lax
from jax.experimental import pallas as pl
from jax.experimental.pallas import tpu as pltpu
from jax.experimental.pallas import tpu_sc as plsc

N = 50000
NG = 64
H = 64
E = 800000
EP = 802816
NP = 50176
QTR = NP // 4
DUMP = 1024
SLAB = QTR + DUMP
PADN = 50100
NC, NS = 2, 16
CH = 128
K1_IT = EP // (NC * NS) // CH
K3_EDGES = EP // NS
K3_IT = K3_EDGES // CH
ZR = SLAB // NS
WBR = QTR // NS
BN = 512
BE = 512
F32 = jnp.float32

_mesh = plsc.VectorSubcoreMesh(core_axis_name="c", subcore_axis_name="s")


NB = 3
K1_G = K1_IT // NB
K1_REM = K1_IT - K1_G * NB


@functools.partial(
    pl.kernel,
    out_type=(jax.ShapeDtypeStruct((EP, 128), F32),
              jax.ShapeDtypeStruct((EP, 128), F32)),
    mesh=_mesh,
    scratch_types=[
        pltpu.VMEM((NB * CH,), jnp.int32),
        pltpu.VMEM((NB * CH,), jnp.int32),
        pltpu.VMEM((NB * CH, 128), F32),
        pltpu.VMEM((NB * CH, 128), F32),
        pltpu.SemaphoreType.DMA,
        pltpu.SemaphoreType.DMA,
        pltpu.SemaphoreType.DMA,
    ],
)
def _gather_k(tbl, sidx, didx, gs, gd, sv, dv, rs, rd, semi, semg, semw):
    c = lax.axis_index("c")
    s = lax.axis_index("s")
    wid = s * NC + c
    base0 = wid * (K1_IT * CH)

    def fire_idx(base, nb):
        pltpu.async_copy(sidx.at[pl.ds(base, nb * CH)],
                         sv.at[pl.ds(0, nb * CH)], semi)
        pltpu.async_copy(didx.at[pl.ds(base, nb * CH)],
                         dv.at[pl.ds(0, nb * CH)], semi)

    def wait_idx(base, nb):
        pltpu.make_async_copy(sidx.at[pl.ds(base, nb * CH)],
                              sv.at[pl.ds(0, nb * CH)], semi).wait()
        pltpu.make_async_copy(didx.at[pl.ds(base, nb * CH)],
                              dv.at[pl.ds(0, nb * CH)], semi).wait()

    def group(base, nb, pf_base, pf_nb):
        wait_idx(base, nb)
        gcps = []
        for k in range(nb):
            gcps.append((
                pltpu.async_copy(tbl.at[sv.at[pl.ds(k * CH, CH)]],
                                 rs.at[pl.ds(k * CH, CH)], semg),
                pltpu.async_copy(tbl.at[dv.at[pl.ds(k * CH, CH)]],
                                 rd.at[pl.ds(k * CH, CH)], semg)))
        wcps = []
        for k in range(nb):
            gcps[k][0].wait()
            gcps[k][1].wait()
            wcps.append(pltpu.async_copy(rs.at[pl.ds(k * CH, CH)],
                                         gs.at[pl.ds(base + k * CH, CH)],
                                         semw))
            wcps.append(pltpu.async_copy(rd.at[pl.ds(k * CH, CH)],
                                         gd.at[pl.ds(base + k * CH, CH)],
                                         semw))
        if pf_nb:
            fire_idx(pf_base, pf_nb)
        for cp in wcps:
            cp.wait()

    fire_idx(base0, NB)

    def body(g, carry):
        base = base0 + g * NB * CH
        group(base, NB, base + NB * CH, NB)
        return carry

    lax.fori_loop(0, K1_G - 1, body, 0)
    base_l = base0 + (K1_G - 1) * NB * CH
    base_r = base0 + K1_G * NB * CH
    group(base_l, NB, base_r, K1_REM)
    if K1_REM:
        group(base_r, K1_REM, 0, 0)


CHS = 64
K3_C = K3_EDGES // CHS
K3_P = K3_C // 2 - 1


def _scatter_body(msg, idx, zrows, aggr, dvs, lvs, mrs, slab,
                  seml, semq, load_rows):
    c = lax.axis_index("c")
    s = lax.axis_index("s")
    base0 = s * K3_EDGES

    def chunk_load(sl, ci):
        base = base0 + ci * CHS
        cps = [pltpu.async_copy(idx.at[pl.ds(base, CHS)], dvs[sl], seml[sl])]
        if load_rows:
            cps.append(pltpu.async_copy(msg.at[pl.ds(base, CHS)],
                                        mrs[sl], seml[sl]))
        return cps

    def chunk_proc(sl, ci, q_base, loads):
        base = base0 + ci * CHS
        for cp in loads:
            cp.wait()
        for j in range(CHS // 16):
            v = dvs[sl][pl.ds(j * 16, 16)]
            loc = v - q_base
            inr = (v >= q_base) & (v < q_base + QTR)
            spread = QTR + ((base + j * 16
                             + lax.iota(jnp.int32, 16)) & (DUMP - 1))
            lvs[sl][pl.ds(j * 16, 16)] = jnp.where(inr, loc, spread)
        return pltpu.async_copy(mrs[sl], slab.at[lvs[sl]], semq[sl],
                                add=True)

    for p in range(2):
        q_base = (2 * c + p) * QTR
        pltpu.sync_copy(zrows, slab.at[pl.ds(s * ZR, ZR)])
        plsc.subcore_barrier()

        l0 = chunk_load(0, 0)

        def body(it2, carry):
            a = 2 * it2
            l1 = chunk_load(1, a + 1)
            q0 = chunk_proc(0, a, q_base, l0)
            q1 = chunk_proc(1, a + 1, q_base, l1)
            q0.wait()
            l0n = chunk_load(0, a + 2)
            q1.wait()
            return carry

        lax.fori_loop(0, K3_P, body, 0)
        l1 = chunk_load(1, K3_C - 1)
        q0 = chunk_proc(0, K3_C - 2, q_base, l0)
        q1 = chunk_proc(1, K3_C - 1, q_base, l1)
        q0.wait()
        q1.wait()
        plsc.subcore_barrier()
        pltpu.sync_copy(slab.at[pl.ds(s * WBR, WBR)],
                        aggr.at[pl.ds(q_base + s * WBR, WBR)])
        plsc.subcore_barrier()


_scatter_scratch = [
    pltpu.VMEM((CHS,), jnp.int32),
    pltpu.VMEM((CHS,), jnp.int32),
    pltpu.VMEM((CHS,), jnp.int32),
    pltpu.VMEM((CHS,), jnp.int32),
    pltpu.VMEM((CHS, 128), F32),
    pltpu.VMEM((CHS, 128), F32),
    pltpu.VMEM_SHARED((SLAB, 128), F32),
    pltpu.SemaphoreType.DMA,
    pltpu.SemaphoreType.DMA,
    pltpu.SemaphoreType.DMA,
    pltpu.SemaphoreType.DMA,
]


@functools.partial(
    pl.kernel,
    out_type=jax.ShapeDtypeStruct((NP, 128), F32),
    mesh=_mesh,
    scratch_types=_scatter_scratch,
)
def _scatter_k(msg, didx, zrows, aggr, dv0, dv1, lv0, lv1, mr0, mr1, slab,
               seml0, seml1, semq0, semq1):
    _scatter_body(msg, didx, zrows, aggr, (dv0, dv1), (lv0, lv1), (mr0, mr1),
                  slab, (seml0, seml1), (semq0, semq1), True)


@functools.partial(
    pl.kernel,
    out_type=jax.ShapeDtypeStruct((NP, 128), F32),
    mesh=_mesh,
    scratch_types=_scatter_scratch,
)
def _counts_k(ones128, sidx, zrows, cnt, dv0, dv1, lv0, lv1, mr0, mr1, slab,
              seml0, seml1, semq0, semq1):
    pltpu.sync_copy(ones128, mr0)
    pltpu.sync_copy(ones128, mr1)
    _scatter_body(ones128, sidx, zrows, cnt, (dv0, dv1), (lv0, lv1),
                  (mr0, mr1), slab, (seml0, seml1), (semq0, semq1), False)


def _ln(x, g, b):
    mu = jnp.mean(x, axis=-1, keepdims=True)
    var = jnp.mean((x - mu) ** 2, axis=-1, keepdims=True)
    return (x - mu) / jnp.sqrt(var + 1e-5) * g + b


def _silu(x):
    return x * jax.nn.sigmoid(x)


def _dot(a, b):
    return jnp.dot(a, b, preferred_element_type=F32,
                   precision=lax.Precision.HIGHEST)


def _edge_body(emit_ea, gd, gs, ea, whd, whs, wd2, wea, b0, g0, bb0,
               w1, b1, g1, bb1, xw0, xb0, xg, xbb, xw18, xb18, *outs):
    hd = gd[:, 0:64]
    hs = gs[:, 0:64]
    dx = gd[:, 64:72] - gs[:, 64:72]
    d2 = jnp.sum(dx * dx, axis=1, keepdims=True)
    pre = (_dot(hd, whd[...]) + _dot(hs, whs[...]) + d2 * wd2[...]
           + _dot(ea[...], wea[...]) + b0[...])
    e1 = _silu(_ln(pre, g0[...], bb0[...]))
    mh = _silu(_ln(_dot(e1, w1[...]) + b1[...], g1[...], bb1[...]))
    t3 = _silu(_ln(_dot(mh, xw0[...]) + xb0[...], xg[...], xbb[...]))
    px = _dot(t3, xw18[...]) + xb18[...]
    mx = dx * px
    outs[0][...] = jnp.concatenate([mh, mx, jnp.zeros((BE, 56), F32)], axis=1)
    if emit_ea:
        outs[1][...] = jnp.concatenate(
            [ea[:, 0:2], d2, jnp.zeros((BE, 5), F32)], axis=1)


def _edge_mlp(emit_ea, gd, gs, ea, weights):
    full = lambda r, c: pl.BlockSpec((r, c), lambda i: (0, 0))
    eblk = lambda c: pl.BlockSpec((BE, c), lambda i: (i, 0))
    wspecs = [full(64, 64), full(64, 64), full(1, 64), full(8, 64),
              full(1, 64), full(1, 64), full(1, 64),
              full(64, 64), full(1, 64), full(1, 64), full(1, 64),
              full(64, 64), full(1, 64), full(1, 64), full(1, 64),
              full(64, 8), full(1, 8)]
    out_shape = [jax.ShapeDtypeStruct((EP, 128), F32)]
    out_specs = [eblk(128)]
    if emit_ea:
        out_shape.append(jax.ShapeDtypeStruct((EP, 8), F32))
        out_specs.append(eblk(8))
    res = pl.pallas_call(
        functools.partial(_edge_body, emit_ea),
        grid=(EP // BE,),
        in_specs=[eblk(128), eblk(128), eblk(8)] + wspecs,
        out_specs=out_specs,
        out_shape=out_shape,
    )(gd, gs, ea, *weights)
    return res if emit_ea else res[0]


def _node_body(tin, ain, cin, hw0a, hw0b, hb0, hg, hbb, hw1, hb1, tout):
    h = tin[:, 0:64]
    mh = ain[:, 0:64]
    pre = _dot(h, hw0a[...]) + _dot(mh, hw0b[...]) + hb0[...]
    ho = _dot(_silu(_ln(pre, hg[...], hbb[...])), hw1[...]) + hb1[...] + h
    cmax = jnp.maximum(cin[:, 0:1], 1.0)
    xo = tin[:, 64:72] + ain[:, 64:72] / cmax
    tout[...] = jnp.concatenate([ho, xo, jnp.zeros((BN, 56), F32)], axis=1)


def _node_update(tcur, aggr, cnt, weights):
    full = lambda r, c: pl.BlockSpec((r, c), lambda i: (0, 0))
    nblk = lambda c: pl.BlockSpec((BN, c), lambda i: (i, 0))
    wspecs = [full(64, 64), full(64, 64), full(1, 64), full(1, 64),
              full(1, 64), full(64, 64), full(1, 64)]
    return pl.pallas_call(
        _node_body,
        grid=(NP // BN,),
        in_specs=[nblk(128), nblk(128), nblk(128)] + wspecs,
        out_specs=nblk(128),
        out_shape=jax.ShapeDtypeStruct((NP, 128), F32),
    )(tcur, aggr, cnt, *weights)


def _embed_body(na, bt, posb, tcol, ewna, ewt, eb, tout):
    oh = (bt[...] == lax.broadcasted_iota(jnp.int32, (1, NG), 1)).astype(F32)
    tb = _dot(oh, tcol[...])
    h0 = _dot(na[...], ewna[...]) + tb * ewt[...] + eb[...]
    tout[...] = jnp.concatenate([h0, posb[...], jnp.zeros((BN, 61), F32)],
                                axis=1)


def _embed(nap, bt, posp, tcol, ewna, ewt, eb):
    full = lambda r, c: pl.BlockSpec((r, c), lambda i: (0, 0))
    nblk = lambda c: pl.BlockSpec((BN, c), lambda i: (i, 0))
    return pl.pallas_call(
        _embed_body,
        grid=(NP // BN,),
        in_specs=[nblk(10), nblk(1), nblk(3), full(NG, 1), full(10, 64),
                  full(1, 64), full(1, 64)],
        out_specs=nblk(128),
        out_shape=jax.ShapeDtypeStruct((NP, 128), F32),
    )(nap, bt, posp, tcol, ewna, ewt, eb)


def _segsum_body(t4, t0, bt, out):
    i = pl.program_id(0)
    d8 = t4[:, 64:72] - t0[:, 64:72]
    rhs = jnp.concatenate([d8[:, 0:7], jnp.ones((BN, 1), F32)], axis=1)
    oh = (bt[...] == lax.broadcasted_iota(jnp.int32, (1, NG), 1)).astype(F32)
    part = lax.dot_general(oh, rhs, (((0,), (0,)), ((), ())),
                           preferred_element_type=F32)

    @pl.when(i == 0)
    def _():
        out[...] = part

    @pl.when(i > 0)
    def _():
        out[...] += part


def _segsum(t4, t0, bt):
    nblk = lambda c: pl.BlockSpec((BN, c), lambda i: (i, 0))
    return pl.pallas_call(
        _segsum_body,
        grid=(NP // BN,),
        in_specs=[nblk(128), nblk(128), nblk(1)],
        out_specs=pl.BlockSpec((NG, 8), lambda i: (0, 0)),
        out_shape=jax.ShapeDtypeStruct((NG, 8), F32),
    )(t4, t0, bt)


def _correct_body(t4, t0, bt, seg, out):
    d8 = t4[:, 64:72] - t0[:, 64:72]
    s = seg[...]
    g = s / jnp.maximum(s[:, 7:8], 1.0)
    oh = (bt[...] == lax.broadcasted_iota(jnp.int32, (1, NG), 1)).astype(F32)
    corr = _dot(oh, g)
    out[...] = d8 - corr


def _correct(t4, t0, bt, seg):
    nblk = lambda c: pl.BlockSpec((BN, c), lambda i: (i, 0))
    return pl.pallas_call(
        _correct_body,
        grid=(NP // BN,),
        in_specs=[nblk(128), nblk(128), nblk(1),
                  pl.BlockSpec((NG, 8), lambda i: (0, 0))],
        out_specs=nblk(8),
        out_shape=jax.ShapeDtypeStruct((NP, 8), F32),
    )(t4, t0, bt, seg)


def kernel(t, positions, edge_index, batch, node_attrs, edge_attrs, params):
    src = edge_index[0].astype(jnp.int32)
    dst = edge_index[1].astype(jnp.int32)
    pad_i = jnp.full((EP - E,), PADN, jnp.int32)
    sidx = jnp.concatenate([src, pad_i])
    didx = jnp.concatenate([dst, pad_i])
    bonds = jax.nn.one_hot(edge_attrs[:, 1], 2, dtype=F32)
    bonds8 = jnp.zeros((EP, 8), F32).at[:E, 0:2].set(bonds)
    bt = jnp.concatenate(
        [batch.astype(jnp.int32), jnp.full((NP - N,), NG, jnp.int32)]
    ).reshape(NP, 1)
    nap = jnp.zeros((NP, 10), F32).at[:N].set(node_attrs)
    posp = jnp.zeros((NP, 3), F32).at[:N].set(positions)
    zrows = jnp.zeros((ZR, 128), F32)
    ones128 = jnp.zeros((CHS, 128), F32).at[:, 0].set(1.0)
    p = params
    tcol = t.reshape(NG, 1).astype(F32)

    t0 = _embed(nap, bt, posp, tcol, p['emb_w'][0:10], p['emb_w'][10:11],
                p['emb_b'].reshape(1, H))
    cnt = _counts_k(ones128, sidx, zrows)

    tc = t0
    ea8 = bonds8
    for li, lp in enumerate(p['layers']):
        gs, gd = _gather_k(tc, sidx, didx)
        r = lambda a: a.reshape(1, -1)
        if li == 0:
            wd2 = lp['e_w0'][128:129] + lp['e_w0'][131:132]
            wea = jnp.zeros((8, 64), F32).at[0:2].set(lp['e_w0'][129:131])
        else:
            wd2 = lp['e_w0'][128:129]
            wea = (jnp.zeros((8, 64), F32).at[0:2].set(lp['e_w0'][129:131])
                   .at[2].set(lp['e_w0'][131]))
        ew = [lp['e_w0'][0:64], lp['e_w0'][64:128], wd2, wea,
              r(lp['e_b0']), r(lp['e_g0']), r(lp['e_bb0']),
              lp['e_w1'], r(lp['e_b1']), r(lp['e_g1']), r(lp['e_bb1']),
              lp['x_w0'], r(lp['x_b0']), r(lp['x_g']), r(lp['x_bb']),
              jnp.tile(lp['x_w1'], (1, 8)),
              jnp.broadcast_to(lp['x_b1'].reshape(1, 1), (1, 8))]
        if li == 0:
            msg, ea8 = _edge_mlp(True, gd, gs, bonds8, ew)
        else:
            msg = _edge_mlp(False, gd, gs, ea8, ew)
        aggr = _scatter_k(msg, didx, zrows)
        nw = [lp['h_w0'][0:64], lp['h_w0'][64:128], r(lp['h_b0']),
              r(lp['h_g']), r(lp['h_bb']), lp['h_w1'], r(lp['h_b1'])]
        tc = _node_update(tc, aggr, cnt, nw)

    seg = _segsum(tc, t0, bt)
    outp = _correct(tc, t0, bt, seg)
    return outp[:N, 0:3]


# speedup vs baseline: 1.6086x; 1.6086x over previous
import functools

import jax
import jax.numpy as jnp
from jax import lax
from jax.experimental import pallas as pl
from jax.experimental.pallas import tpu as pltpu
from jax.experimental.pallas import tpu_sc as plsc

N = 50000
NG = 64
H = 64
E = 800000
EP = 802816
NP = 50176
QTR = NP // 4
DUMP = 1024
SLAB = QTR + DUMP
PADN = 50100
NC, NS = 2, 16
CH = 128
K1_IT = EP // (NC * NS) // CH
K3_EDGES = EP // NS
K3_IT = K3_EDGES // CH
ZR = SLAB // NS
WBR = QTR // NS
BN = 512
BE = 512
F32 = jnp.float32

_mesh = plsc.VectorSubcoreMesh(core_axis_name="c", subcore_axis_name="s")


NB = 3
K1_G = K1_IT // NB
K1_REM = K1_IT - K1_G * NB


@functools.partial(
    pl.kernel,
    out_type=(jax.ShapeDtypeStruct((EP, 128), F32),
              jax.ShapeDtypeStruct((EP, 128), F32)),
    mesh=_mesh,
    scratch_types=[
        pltpu.VMEM((NB * CH,), jnp.int32),
        pltpu.VMEM((NB * CH,), jnp.int32),
        pltpu.VMEM((NB * CH, 128), F32),
        pltpu.VMEM((NB * CH, 128), F32),
        pltpu.SemaphoreType.DMA,
        pltpu.SemaphoreType.DMA,
        pltpu.SemaphoreType.DMA,
    ],
)
def _gather_k(tbl, sidx, didx, gs, gd, sv, dv, rs, rd, semi, semg, semw):
    c = lax.axis_index("c")
    s = lax.axis_index("s")
    wid = s * NC + c
    base0 = wid * (K1_IT * CH)

    def fire_idx(base, nb):
        pltpu.async_copy(sidx.at[pl.ds(base, nb * CH)],
                         sv.at[pl.ds(0, nb * CH)], semi)
        pltpu.async_copy(didx.at[pl.ds(base, nb * CH)],
                         dv.at[pl.ds(0, nb * CH)], semi)

    def wait_idx(base, nb):
        pltpu.make_async_copy(sidx.at[pl.ds(base, nb * CH)],
                              sv.at[pl.ds(0, nb * CH)], semi).wait()
        pltpu.make_async_copy(didx.at[pl.ds(base, nb * CH)],
                              dv.at[pl.ds(0, nb * CH)], semi).wait()

    def group(base, nb, pf_base, pf_nb):
        wait_idx(base, nb)
        gcps = []
        for k in range(nb):
            gcps.append((
                pltpu.async_copy(tbl.at[sv.at[pl.ds(k * CH, CH)]],
                                 rs.at[pl.ds(k * CH, CH)], semg),
                pltpu.async_copy(tbl.at[dv.at[pl.ds(k * CH, CH)]],
                                 rd.at[pl.ds(k * CH, CH)], semg)))
        wcps = []
        for k in range(nb):
            gcps[k][0].wait()
            gcps[k][1].wait()
            wcps.append(pltpu.async_copy(rs.at[pl.ds(k * CH, CH)],
                                         gs.at[pl.ds(base + k * CH, CH)],
                                         semw))
            wcps.append(pltpu.async_copy(rd.at[pl.ds(k * CH, CH)],
                                         gd.at[pl.ds(base + k * CH, CH)],
                                         semw))
        if pf_nb:
            fire_idx(pf_base, pf_nb)
        for cp in wcps:
            cp.wait()

    fire_idx(base0, NB)

    def body(g, carry):
        base = base0 + g * NB * CH
        group(base, NB, base + NB * CH, NB)
        return carry

    lax.fori_loop(0, K1_G - 1, body, 0)
    base_l = base0 + (K1_G - 1) * NB * CH
    base_r = base0 + K1_G * NB * CH
    group(base_l, NB, base_r, K1_REM)
    if K1_REM:
        group(base_r, K1_REM, 0, 0)


CHS = 64
K3_C = K3_EDGES // CHS
K3_P = K3_C // 2 - 1


def _scatter_body(msg, idx, zrows, aggr, dvs, lvs, mrs, slab,
                  seml, semq, load_rows):
    c = lax.axis_index("c")
    s = lax.axis_index("s")
    base0 = s * K3_EDGES

    def chunk_load(sl, ci):
        base = base0 + ci * CHS
        cps = [pltpu.async_copy(idx.at[pl.ds(base, CHS)], dvs[sl], seml[sl])]
        if load_rows:
            cps.append(pltpu.async_copy(msg.at[pl.ds(base, CHS)],
                                        mrs[sl], seml[sl]))
        return cps

    def chunk_proc(sl, ci, q_base, loads):
        base = base0 + ci * CHS
        for cp in loads:
            cp.wait()
        for j in range(CHS // 16):
            v = dvs[sl][pl.ds(j * 16, 16)]
            loc = v - q_base
            inr = (v >= q_base) & (v < q_base + QTR)
            spread = QTR + ((base + j * 16
                             + lax.iota(jnp.int32, 16)) & (DUMP - 1))
            lvs[sl][pl.ds(j * 16, 16)] = jnp.where(inr, loc, spread)
        return pltpu.async_copy(mrs[sl], slab.at[lvs[sl]], semq[sl],
                                add=True)

    for p in range(2):
        q_base = (2 * c + p) * QTR
        pltpu.sync_copy(zrows, slab.at[pl.ds(s * ZR, ZR)])
        plsc.subcore_barrier()

        l0 = chunk_load(0, 0)

        def body(it2, carry):
            a = 2 * it2
            l1 = chunk_load(1, a + 1)
            q0 = chunk_proc(0, a, q_base, l0)
            q1 = chunk_proc(1, a + 1, q_base, l1)
            q0.wait()
            l0n = chunk_load(0, a + 2)
            q1.wait()
            return carry

        lax.fori_loop(0, K3_P, body, 0)
        l1 = chunk_load(1, K3_C - 1)
        q0 = chunk_proc(0, K3_C - 2, q_base, l0)
        q1 = chunk_proc(1, K3_C - 1, q_base, l1)
        q0.wait()
        q1.wait()
        plsc.subcore_barrier()
        pltpu.sync_copy(slab.at[pl.ds(s * WBR, WBR)],
                        aggr.at[pl.ds(q_base + s * WBR, WBR)])
        plsc.subcore_barrier()


_scatter_scratch = [
    pltpu.VMEM((CHS,), jnp.int32),
    pltpu.VMEM((CHS,), jnp.int32),
    pltpu.VMEM((CHS,), jnp.int32),
    pltpu.VMEM((CHS,), jnp.int32),
    pltpu.VMEM((CHS, 128), F32),
    pltpu.VMEM((CHS, 128), F32),
    pltpu.VMEM_SHARED((SLAB, 128), F32),
    pltpu.SemaphoreType.DMA,
    pltpu.SemaphoreType.DMA,
    pltpu.SemaphoreType.DMA,
    pltpu.SemaphoreType.DMA,
]


@functools.partial(
    pl.kernel,
    out_type=jax.ShapeDtypeStruct((NP, 128), F32),
    mesh=_mesh,
    scratch_types=_scatter_scratch,
)
def _scatter_k(msg, didx, zrows, aggr, dv0, dv1, lv0, lv1, mr0, mr1, slab,
               seml0, seml1, semq0, semq1):
    _scatter_body(msg, didx, zrows, aggr, (dv0, dv1), (lv0, lv1), (mr0, mr1),
                  slab, (seml0, seml1), (semq0, semq1), True)


@functools.partial(
    pl.kernel,
    out_type=jax.ShapeDtypeStruct((NP, 128), F32),
    mesh=_mesh,
    scratch_types=_scatter_scratch,
)
def _counts_k(ones128, sidx, zrows, cnt, dv0, dv1, lv0, lv1, mr0, mr1, slab,
              seml0, seml1, semq0, semq1):
    pltpu.sync_copy(ones128, mr0)
    pltpu.sync_copy(ones128, mr1)
    _scatter_body(ones128, sidx, zrows, cnt, (dv0, dv1), (lv0, lv1),
                  (mr0, mr1), slab, (seml0, seml1), (semq0, semq1), False)


def _ln(x, g, b):
    mu = jnp.mean(x, axis=-1, keepdims=True)
    var = jnp.mean((x - mu) ** 2, axis=-1, keepdims=True)
    return (x - mu) / jnp.sqrt(var + 1e-5) * g + b


def _silu(x):
    return x * jax.nn.sigmoid(x)


def _dot(a, b):
    return jnp.dot(a, b, preferred_element_type=F32)


def _edge_body(emit_ea, gd, gs, ea, whd, whs, wd2, wea, b0, g0, bb0,
               w1, b1, g1, bb1, xw0, xb0, xg, xbb, xw18, xb18, *outs):
    hd = gd[:, 0:64]
    hs = gs[:, 0:64]
    dx = gd[:, 64:72] - gs[:, 64:72]
    d2 = jnp.sum(dx * dx, axis=1, keepdims=True)
    pre = (_dot(hd, whd[...]) + _dot(hs, whs[...]) + d2 * wd2[...]
           + _dot(ea[...], wea[...]) + b0[...])
    e1 = _silu(_ln(pre, g0[...], bb0[...]))
    mh = _silu(_ln(_dot(e1, w1[...]) + b1[...], g1[...], bb1[...]))
    t3 = _silu(_ln(_dot(mh, xw0[...]) + xb0[...], xg[...], xbb[...]))
    px = _dot(t3, xw18[...]) + xb18[...]
    mx = dx * px
    outs[0][...] = jnp.concatenate([mh, mx, jnp.zeros((BE, 56), F32)], axis=1)
    if emit_ea:
        outs[1][...] = jnp.concatenate(
            [ea[:, 0:2], d2, jnp.zeros((BE, 5), F32)], axis=1)


def _edge_mlp(emit_ea, gd, gs, ea, weights):
    full = lambda r, c: pl.BlockSpec((r, c), lambda i: (0, 0))
    eblk = lambda c: pl.BlockSpec((BE, c), lambda i: (i, 0))
    wspecs = [full(64, 64), full(64, 64), full(1, 64), full(8, 64),
              full(1, 64), full(1, 64), full(1, 64),
              full(64, 64), full(1, 64), full(1, 64), full(1, 64),
              full(64, 64), full(1, 64), full(1, 64), full(1, 64),
              full(64, 8), full(1, 8)]
    out_shape = [jax.ShapeDtypeStruct((EP, 128), F32)]
    out_specs = [eblk(128)]
    if emit_ea:
        out_shape.append(jax.ShapeDtypeStruct((EP, 8), F32))
        out_specs.append(eblk(8))
    res = pl.pallas_call(
        functools.partial(_edge_body, emit_ea),
        grid=(EP // BE,),
        in_specs=[eblk(128), eblk(128), eblk(8)] + wspecs,
        out_specs=out_specs,
        out_shape=out_shape,
    )(gd, gs, ea, *weights)
    return res if emit_ea else res[0]


def _node_body(tin, ain, cin, hw0a, hw0b, hb0, hg, hbb, hw1, hb1, tout):
    h = tin[:, 0:64]
    mh = ain[:, 0:64]
    pre = _dot(h, hw0a[...]) + _dot(mh, hw0b[...]) + hb0[...]
    ho = _dot(_silu(_ln(pre, hg[...], hbb[...])), hw1[...]) + hb1[...] + h
    cmax = jnp.maximum(cin[:, 0:1], 1.0)
    xo = tin[:, 64:72] + ain[:, 64:72] / cmax
    tout[...] = jnp.concatenate([ho, xo, jnp.zeros((BN, 56), F32)], axis=1)


def _node_update(tcur, aggr, cnt, weights):
    full = lambda r, c: pl.BlockSpec((r, c), lambda i: (0, 0))
    nblk = lambda c: pl.BlockSpec((BN, c), lambda i: (i, 0))
    wspecs = [full(64, 64), full(64, 64), full(1, 64), full(1, 64),
              full(1, 64), full(64, 64), full(1, 64)]
    return pl.pallas_call(
        _node_body,
        grid=(NP // BN,),
        in_specs=[nblk(128), nblk(128), nblk(128)] + wspecs,
        out_specs=nblk(128),
        out_shape=jax.ShapeDtypeStruct((NP, 128), F32),
    )(tcur, aggr, cnt, *weights)


def _embed_body(na, bt, posb, tcol, ewna, ewt, eb, tout):
    oh = (bt[...] == lax.broadcasted_iota(jnp.int32, (1, NG), 1)).astype(F32)
    tb = _dot(oh, tcol[...])
    h0 = _dot(na[...], ewna[...]) + tb * ewt[...] + eb[...]
    tout[...] = jnp.concatenate([h0, posb[...], jnp.zeros((BN, 61), F32)],
                                axis=1)


def _embed(nap, bt, posp, tcol, ewna, ewt, eb):
    full = lambda r, c: pl.BlockSpec((r, c), lambda i: (0, 0))
    nblk = lambda c: pl.BlockSpec((BN, c), lambda i: (i, 0))
    return pl.pallas_call(
        _embed_body,
        grid=(NP // BN,),
        in_specs=[nblk(10), nblk(1), nblk(3), full(NG, 1), full(10, 64),
                  full(1, 64), full(1, 64)],
        out_specs=nblk(128),
        out_shape=jax.ShapeDtypeStruct((NP, 128), F32),
    )(nap, bt, posp, tcol, ewna, ewt, eb)


def _segsum_body(t4, t0, bt, out):
    i = pl.program_id(0)
    d8 = t4[:, 64:72] - t0[:, 64:72]
    rhs = jnp.concatenate([d8[:, 0:7], jnp.ones((BN, 1), F32)], axis=1)
    oh = (bt[...] == lax.broadcasted_iota(jnp.int32, (1, NG), 1)).astype(F32)
    part = lax.dot_general(oh, rhs, (((0,), (0,)), ((), ())),
                           preferred_element_type=F32)

    @pl.when(i == 0)
    def _():
        out[...] = part

    @pl.when(i > 0)
    def _():
        out[...] += part


def _segsum(t4, t0, bt):
    nblk = lambda c: pl.BlockSpec((BN, c), lambda i: (i, 0))
    return pl.pallas_call(
        _segsum_body,
        grid=(NP // BN,),
        in_specs=[nblk(128), nblk(128), nblk(1)],
        out_specs=pl.BlockSpec((NG, 8), lambda i: (0, 0)),
        out_shape=jax.ShapeDtypeStruct((NG, 8), F32),
    )(t4, t0, bt)


def _correct_body(t4, t0, bt, seg, out):
    d8 = t4[:, 64:72] - t0[:, 64:72]
    s = seg[...]
    g = s / jnp.maximum(s[:, 7:8], 1.0)
    oh = (bt[...] == lax.broadcasted_iota(jnp.int32, (1, NG), 1)).astype(F32)
    corr = _dot(oh, g)
    out[...] = d8 - corr


def _correct(t4, t0, bt, seg):
    nblk = lambda c: pl.BlockSpec((BN, c), lambda i: (i, 0))
    return pl.pallas_call(
        _correct_body,
        grid=(NP // BN,),
        in_specs=[nblk(128), nblk(128), nblk(1),
                  pl.BlockSpec((NG, 8), lambda i: (0, 0))],
        out_specs=nblk(8),
        out_shape=jax.ShapeDtypeStruct((NP, 8), F32),
    )(t4, t0, bt, seg)


def kernel(t, positions, edge_index, batch, node_attrs, edge_attrs, params):
    src = edge_index[0].astype(jnp.int32)
    dst = edge_index[1].astype(jnp.int32)
    pad_i = jnp.full((EP - E,), PADN, jnp.int32)
    sidx = jnp.concatenate([src, pad_i])
    didx = jnp.concatenate([dst, pad_i])
    bonds = jax.nn.one_hot(edge_attrs[:, 1], 2, dtype=F32)
    bonds8 = jnp.zeros((EP, 8), F32).at[:E, 0:2].set(bonds)
    bt = jnp.concatenate(
        [batch.astype(jnp.int32), jnp.full((NP - N,), NG, jnp.int32)]
    ).reshape(NP, 1)
    nap = jnp.zeros((NP, 10), F32).at[:N].set(node_attrs)
    posp = jnp.zeros((NP, 3), F32).at[:N].set(positions)
    zrows = jnp.zeros((ZR, 128), F32)
    ones128 = jnp.zeros((CHS, 128), F32).at[:, 0].set(1.0)
    p = params
    tcol = t.reshape(NG, 1).astype(F32)

    t0 = _embed(nap, bt, posp, tcol, p['emb_w'][0:10], p['emb_w'][10:11],
                p['emb_b'].reshape(1, H))
    cnt = _counts_k(ones128, sidx, zrows)

    tc = t0
    ea8 = bonds8
    for li, lp in enumerate(p['layers']):
        gs, gd = _gather_k(tc, sidx, didx)
        r = lambda a: a.reshape(1, -1)
        if li == 0:
            wd2 = lp['e_w0'][128:129] + lp['e_w0'][131:132]
            wea = jnp.zeros((8, 64), F32).at[0:2].set(lp['e_w0'][129:131])
        else:
            wd2 = lp['e_w0'][128:129]
            wea = (jnp.zeros((8, 64), F32).at[0:2].set(lp['e_w0'][129:131])
                   .at[2].set(lp['e_w0'][131]))
        ew = [lp['e_w0'][0:64], lp['e_w0'][64:128], wd2, wea,
              r(lp['e_b0']), r(lp['e_g0']), r(lp['e_bb0']),
              lp['e_w1'], r(lp['e_b1']), r(lp['e_g1']), r(lp['e_bb1']),
              lp['x_w0'], r(lp['x_b0']), r(lp['x_g']), r(lp['x_bb']),
              jnp.tile(lp['x_w1'], (1, 8)),
              jnp.broadcast_to(lp['x_b1'].reshape(1, 1), (1, 8))]
        if li == 0:
            msg, ea8 = _edge_mlp(True, gd, gs, bonds8, ew)
        else:
            msg = _edge_mlp(False, gd, gs, ea8, ew)
        aggr = _scatter_k(msg, didx, zrows)
        nw = [lp['h_w0'][0:64], lp['h_w0'][64:128], r(lp['h_b0']),
              r(lp['h_g']), r(lp['h_bb']), lp['h_w1'], r(lp['h_b1'])]
        tc = _node_update(tc, aggr, cnt, nw)

    seg = _segsum(tc, t0, bt)
    outp = _correct(tc, t0, bt, seg)
    return outp[:N, 0:3]
